# Initial kernel scaffold; baseline (speedup 1.0000x reference)
import jax
import jax.numpy as jnp
from jax.experimental import pallas as pl


def kernel(features, edge_index, norm, W1, W2):
    raise NotImplementedError("write your pallas kernel here")


# speedup vs baseline: 4.2663x; 4.2663x over previous
import functools

import jax
import jax.numpy as jnp
from jax import lax
from jax.experimental import pallas as pl
from jax.experimental.pallas import tpu as pltpu
from jax.experimental.pallas import tpu_sc as plsc

NC = 2
NS = 16
NW = NC * NS
CH = 128


def _stage1_body(x_ref, w_ref, n_ref, o_ref):
    h = jnp.dot(x_ref[...], w_ref[...], preferred_element_type=jnp.float32)
    o_ref[...] = h * n_ref[...]


def _tc_stage1(x, w, norm, block_rows=1000):
    n_rows, d_in = x.shape
    d_out = w.shape[1]
    grid = (n_rows // block_rows,)
    return pl.pallas_call(
        _stage1_body,
        grid=grid,
        in_specs=[
            pl.BlockSpec((block_rows, d_in), lambda i: (i, 0)),
            pl.BlockSpec((d_in, d_out), lambda i: (0, 0)),
            pl.BlockSpec((block_rows, 1), lambda i: (i, 0)),
        ],
        out_specs=pl.BlockSpec((block_rows, d_out), lambda i: (i, 0)),
        out_shape=jax.ShapeDtypeStruct((n_rows, d_out), jnp.float32),
    )(x, w, norm)


def _stage2_body(p0_ref, p1_ref, n_ref, w_ref, o_ref):
    nrm = n_ref[...]
    h = jax.nn.relu((p0_ref[...] + p1_ref[...]) * nrm)
    o_ref[...] = jnp.dot(h, w_ref[...], preferred_element_type=jnp.float32) * nrm


def _tc_stage2(p0, p1, norm, w, block_rows=1000):
    n_rows, d_in = p0.shape
    d_out = w.shape[1]
    grid = (n_rows // block_rows,)
    return pl.pallas_call(
        _stage2_body,
        grid=grid,
        in_specs=[
            pl.BlockSpec((block_rows, d_in), lambda i: (i, 0)),
            pl.BlockSpec((block_rows, d_in), lambda i: (i, 0)),
            pl.BlockSpec((block_rows, 1), lambda i: (i, 0)),
            pl.BlockSpec((d_in, d_out), lambda i: (0, 0)),
        ],
        out_specs=pl.BlockSpec((block_rows, d_out), lambda i: (i, 0)),
        out_shape=jax.ShapeDtypeStruct((n_rows, d_out), jnp.float32),
    )(p0, p1, norm, w)


def _stage3_body(q0_ref, q1_ref, n_ref, o_ref):
    o_ref[...] = (q0_ref[...] + q1_ref[...]) * n_ref[...]


def _tc_stage3(q0, q1, norm, block_rows=1000):
    n_rows, d = q0.shape
    grid = (n_rows // block_rows,)
    return pl.pallas_call(
        _stage3_body,
        grid=grid,
        in_specs=[
            pl.BlockSpec((block_rows, d), lambda i: (i, 0)),
            pl.BlockSpec((block_rows, d), lambda i: (i, 0)),
            pl.BlockSpec((block_rows, 1), lambda i: (i, 0)),
        ],
        out_specs=pl.BlockSpec((block_rows, d), lambda i: (i, 0)),
        out_shape=jax.ShapeDtypeStruct((n_rows, d), jnp.float32),
    )(q0, q1, norm)


@functools.cache
def _make_sc_segsum(n_nodes, d, chunks_per_tile, n_pad):
    rows_per_tile_pad = n_pad // NS
    rows_per_tile_out = (n_nodes // NS) // 8 * 8
    tail_start = rows_per_tile_out * NS
    tail_rows = n_nodes - tail_start
    mesh = plsc.VectorSubcoreMesh(core_axis_name="c", subcore_axis_name="s")

    @functools.partial(
        pl.kernel,
        out_type=jax.ShapeDtypeStruct((NC, n_nodes, d), jnp.float32),
        mesh=mesh,
        scratch_types=[
            pltpu.VMEM((CH,), jnp.int32),
            pltpu.VMEM((CH,), jnp.int32),
            pltpu.VMEM((CH, d), jnp.float32),
            pltpu.VMEM_SHARED((n_pad, d), jnp.float32),
            pltpu.SemaphoreType.DMA,
        ],
        compiler_params=pltpu.CompilerParams(use_tc_tiling_on_sc=False),
    )
    def segsum(h_hbm, src_hbm, dst_hbm, out_hbm, sidx, didx, rows, acc, sem):
        cid = lax.axis_index("c")
        sid = lax.axis_index("s")
        wid = sid * NC + cid

        zeros16 = jnp.zeros((16,), jnp.float32)

        @pl.loop(0, CH)
        def _zero_rows(r):
            for c in range(d // 16):
                rows[r, pl.ds(c * 16, 16)] = zeros16

        for t in range(rows_per_tile_pad // CH):
            pltpu.sync_copy(rows, acc.at[pl.ds(sid * rows_per_tile_pad + t * CH, CH)])

        plsc.subcore_barrier()

        @pl.loop(0, chunks_per_tile)
        def _edges(j):
            base = pl.multiple_of((wid * chunks_per_tile + j) * CH, CH)
            pltpu.sync_copy(src_hbm.at[pl.ds(base, CH)], sidx)
            pltpu.sync_copy(dst_hbm.at[pl.ds(base, CH)], didx)
            pltpu.async_copy(h_hbm.at[sidx], rows, sem).wait()
            pltpu.sync_copy(rows, acc.at[didx], add=True)

        plsc.subcore_barrier()

        pltpu.sync_copy(
            acc.at[pl.ds(sid * rows_per_tile_out, rows_per_tile_out)],
            out_hbm.at[cid, pl.ds(sid * rows_per_tile_out, rows_per_tile_out)],
        )
        if tail_rows:
            @pl.when(sid == 0)
            def _tail():
                pltpu.sync_copy(
                    acc.at[pl.ds(tail_start, tail_rows)],
                    out_hbm.at[cid, pl.ds(tail_start, tail_rows)],
                )

    return segsum


def _sc_segsum(h, src_pad, dst_pad, n_nodes, n_pad):
    d = h.shape[1]
    chunks_per_tile = src_pad.shape[0] // (NW * CH)
    fn = _make_sc_segsum(n_nodes, d, chunks_per_tile, n_pad)
    return fn(h, src_pad, dst_pad)


def kernel(features, edge_index, norm, W1, W2):
    n_nodes = features.shape[0]
    n_edges = edge_index.shape[1]

    chunks_per_tile = -(-n_edges // (NW * CH))
    e_pad = NW * chunks_per_tile * CH
    n_pad = NS * CH * (-(-(n_nodes + 1) // (NS * CH)))
    src = edge_index[0]
    dst = edge_index[1]
    if e_pad != n_edges:
        pad = e_pad - n_edges
        src = jnp.concatenate([src, jnp.zeros((pad,), jnp.int32)])
        dst = jnp.concatenate([dst, jnp.full((pad,), n_nodes, jnp.int32)])

    h1 = _tc_stage1(features, W1, norm)
    p = _sc_segsum(h1, src, dst, n_nodes, n_pad)
    h2 = _tc_stage2(p[0], p[1], norm, W2)
    q = _sc_segsum(h2, src, dst, n_nodes, n_pad)
    return _tc_stage3(q[0], q[1], norm)


# speedup vs baseline: 6.6760x; 1.5648x over previous
import functools

import jax
import jax.numpy as jnp
from jax import lax
from jax.experimental import pallas as pl
from jax.experimental.pallas import tpu as pltpu
from jax.experimental.pallas import tpu_sc as plsc

NC = 2
NS = 16
CH = 128


def _stage1_body(x_ref, w_ref, n_ref, o_ref):
    h = jnp.dot(x_ref[...], w_ref[...], preferred_element_type=jnp.float32)
    h = h * n_ref[...]
    d2 = h.shape[1] // 2
    o_ref[0] = h[:, :d2]
    o_ref[1] = h[:, d2:]


def _tc_stage1(x, w, norm, block_rows=1000):
    n_rows, d_in = x.shape
    d_out = w.shape[1]
    grid = (n_rows // block_rows,)
    return pl.pallas_call(
        _stage1_body,
        grid=grid,
        in_specs=[
            pl.BlockSpec((block_rows, d_in), lambda i: (i, 0)),
            pl.BlockSpec((d_in, d_out), lambda i: (0, 0)),
            pl.BlockSpec((block_rows, 1), lambda i: (i, 0)),
        ],
        out_specs=pl.BlockSpec((2, block_rows, d_out // 2), lambda i: (0, i, 0)),
        out_shape=jax.ShapeDtypeStruct((2, n_rows, d_out // 2), jnp.float32),
    )(x, w, norm)


def _stage2_body(p_ref, n_ref, w_ref, o_ref):
    nrm = n_ref[...]
    h = jnp.concatenate([p_ref[0], p_ref[1]], axis=1)
    h = jax.nn.relu(h * nrm)
    h = jnp.dot(h, w_ref[...], preferred_element_type=jnp.float32) * nrm
    d2 = h.shape[1] // 2
    o_ref[0] = h[:, :d2]
    o_ref[1] = h[:, d2:]


def _tc_stage2(p, norm, w, block_rows=1000):
    n_rows = p.shape[1]
    d_in = 2 * p.shape[2]
    d_out = w.shape[1]
    grid = (n_rows // block_rows,)
    return pl.pallas_call(
        _stage2_body,
        grid=grid,
        in_specs=[
            pl.BlockSpec((2, block_rows, d_in // 2), lambda i: (0, i, 0)),
            pl.BlockSpec((block_rows, 1), lambda i: (i, 0)),
            pl.BlockSpec((d_in, d_out), lambda i: (0, 0)),
        ],
        out_specs=pl.BlockSpec((2, block_rows, d_out // 2), lambda i: (0, i, 0)),
        out_shape=jax.ShapeDtypeStruct((2, n_rows, d_out // 2), jnp.float32),
    )(p, norm, w)


def _stage3_body(q_ref, n_ref, o_ref):
    o_ref[...] = jnp.concatenate([q_ref[0], q_ref[1]], axis=1) * n_ref[...]


def _tc_stage3(q, norm, block_rows=1000):
    n_rows = q.shape[1]
    d = 2 * q.shape[2]
    grid = (n_rows // block_rows,)
    return pl.pallas_call(
        _stage3_body,
        grid=grid,
        in_specs=[
            pl.BlockSpec((2, block_rows, d // 2), lambda i: (0, i, 0)),
            pl.BlockSpec((block_rows, 1), lambda i: (i, 0)),
        ],
        out_specs=pl.BlockSpec((block_rows, d), lambda i: (i, 0)),
        out_shape=jax.ShapeDtypeStruct((n_rows, d), jnp.float32),
    )(q, norm)


@functools.cache
def _make_sc_segsum(n_nodes, d2, chunks_per_tile, n_pad):
    rows_per_tile_pad = n_pad // NS
    rows_per_tile_out = (n_nodes // NS) // 8 * 8
    tail_start = rows_per_tile_out * NS
    tail_rows = n_nodes - tail_start
    cpt = chunks_per_tile
    mesh = plsc.VectorSubcoreMesh(core_axis_name="c", subcore_axis_name="s")

    @functools.partial(
        pl.kernel,
        out_type=jax.ShapeDtypeStruct((NC, n_nodes, d2), jnp.float32),
        mesh=mesh,
        scratch_types=[
            pltpu.VMEM((cpt, CH), jnp.int32),
            pltpu.VMEM((cpt, CH), jnp.int32),
            pltpu.VMEM((2, CH, d2), jnp.float32),
            pltpu.VMEM_SHARED((n_pad, d2), jnp.float32),
            pltpu.SemaphoreType.DMA,
        ],
        compiler_params=pltpu.CompilerParams(use_tc_tiling_on_sc=False),
    )
    def segsum(h_hbm, src_hbm, dst_hbm, out_hbm, sidx, didx, rows, acc, sg):
        cid = lax.axis_index("c")
        sid = lax.axis_index("s")

        pltpu.sync_copy(src_hbm.at[pl.ds(sid * cpt, cpt)], sidx)
        pltpu.sync_copy(dst_hbm.at[pl.ds(sid * cpt, cpt)], didx)

        zeros16 = jnp.zeros((16,), jnp.float32)

        @pl.loop(0, CH)
        def _zero_rows(r):
            for c in range(d2 // 16):
                rows[0, r, pl.ds(c * 16, 16)] = zeros16

        for t in range(rows_per_tile_pad // CH):
            pltpu.sync_copy(rows.at[0],
                            acc.at[pl.ds(sid * rows_per_tile_pad + t * CH, CH)])

        plsc.subcore_barrier()

        @pl.loop(0, cpt + 1)
        def _edges(j):
            jc = jnp.minimum(j, cpt - 1)
            cp = pltpu.make_async_copy(
                h_hbm.at[cid].at[sidx.at[jc]], rows.at[lax.rem(jc, 2)], sg)

            @pl.when(j < cpt)
            def _start_gather():
                cp.start()

            @pl.when(j > 0)
            def _scatter_prev():
                jp = j - 1
                pltpu.sync_copy(rows.at[lax.rem(jp, 2)],
                                acc.at[didx.at[jp]], add=True)

            @pl.when(j < cpt)
            def _wait_gather():
                cp.wait()

        plsc.subcore_barrier()

        pltpu.sync_copy(
            acc.at[pl.ds(sid * rows_per_tile_out, rows_per_tile_out)],
            out_hbm.at[cid, pl.ds(sid * rows_per_tile_out, rows_per_tile_out)],
        )
        if tail_rows:
            @pl.when(sid == 0)
            def _tail():
                pltpu.sync_copy(
                    acc.at[pl.ds(tail_start, tail_rows)],
                    out_hbm.at[cid, pl.ds(tail_start, tail_rows)],
                )

    return segsum


def _sc_segsum(h, src_pad, dst_pad, n_nodes, n_pad):
    d2 = h.shape[2]
    chunks_per_tile = src_pad.shape[0] // NS
    fn = _make_sc_segsum(n_nodes, d2, chunks_per_tile, n_pad)
    return fn(h, src_pad, dst_pad)


def kernel(features, edge_index, norm, W1, W2):
    n_nodes = features.shape[0]
    n_edges = edge_index.shape[1]

    chunks_per_tile = -(-n_edges // (NS * CH))
    e_pad = NS * chunks_per_tile * CH
    n_pad = NS * CH * (-(-(n_nodes + 1) // (NS * CH)))
    src = edge_index[0]
    dst = edge_index[1]
    if e_pad != n_edges:
        pad = e_pad - n_edges
        src = jnp.concatenate([src, jnp.zeros((pad,), jnp.int32)])
        dst = jnp.concatenate([dst, jnp.full((pad,), n_nodes, jnp.int32)])
    src = src.reshape(NS * chunks_per_tile, CH)
    dst = dst.reshape(NS * chunks_per_tile, CH)

    h1 = _tc_stage1(features, W1, norm)
    p = _sc_segsum(h1, src, dst, n_nodes, n_pad)
    h2 = _tc_stage2(p, norm, W2)
    q = _sc_segsum(h2, src, dst, n_nodes, n_pad)
    return _tc_stage3(q, norm)
